# Initial kernel scaffold; baseline (speedup 1.0000x reference)
import jax
import jax.numpy as jnp
from jax.experimental import pallas as pl


def kernel(constraint_features, variable_features, edge_attr, W_ce, b_ce, W_ve, b_ve, W_e, b_e, W_m1, b_m1, W_u1, b_u1, W_m2, b_m2, W_u2, b_u2, W_o1, b_o1, W_o2, b_o2, edge_index, graph_num):
    raise NotImplementedError("write your pallas kernel here")


# speedup vs baseline: 2.2541x; 2.2541x over previous
import functools

import jax
import jax.numpy as jnp
from jax import lax
from jax.experimental import pallas as pl
from jax.experimental.pallas import tpu as pltpu
from jax.experimental.pallas import tpu_sc as plsc

F32 = jnp.float32


def _linrelu(x, w, b, bm):
    m, k = x.shape
    n = w.shape[1]

    def body(x_ref, w_ref, b_ref, o_ref):
        o_ref[...] = jnp.maximum(
            jnp.dot(x_ref[...], w_ref[...], preferred_element_type=F32)
            + b_ref[...], 0.0)

    return pl.pallas_call(
        body,
        grid=(m // bm,),
        in_specs=[
            pl.BlockSpec((bm, k), lambda i: (i, 0)),
            pl.BlockSpec((k, n), lambda i: (0, 0)),
            pl.BlockSpec((1, n), lambda i: (0, 0)),
        ],
        out_specs=pl.BlockSpec((bm, n), lambda i: (i, 0)),
        out_shape=jax.ShapeDtypeStruct((m, n), F32),
    )(x, w, b.reshape(1, n))


def _addlinrelu(x, y, w, b, bm):
    m, k = x.shape
    n = w.shape[1]

    def body(x_ref, y_ref, w_ref, b_ref, o_ref):
        o_ref[...] = jnp.maximum(
            jnp.dot(x_ref[...] + y_ref[...], w_ref[...],
                    preferred_element_type=F32) + b_ref[...], 0.0)

    return pl.pallas_call(
        body,
        grid=(m // bm,),
        in_specs=[
            pl.BlockSpec((bm, k), lambda i: (i, 0)),
            pl.BlockSpec((bm, k), lambda i: (i, 0)),
            pl.BlockSpec((k, n), lambda i: (0, 0)),
            pl.BlockSpec((1, n), lambda i: (0, 0)),
        ],
        out_specs=pl.BlockSpec((bm, n), lambda i: (i, 0)),
        out_shape=jax.ShapeDtypeStruct((m, n), F32),
    )(x, y, w, b.reshape(1, n))


def _edge_stage(g, ea, w_e, b_e, w_m, b_m, bm):
    m, h = g.shape
    de = ea.shape[1]

    def body(g_ref, ea_ref, we_ref, be_ref, wm_ref, bm_ref, o_ref):
        e = jnp.maximum(
            jnp.dot(ea_ref[...], we_ref[...], preferred_element_type=F32)
            + be_ref[...], 0.0)
        z = g_ref[...] + e
        o_ref[...] = jnp.maximum(
            jnp.dot(z, wm_ref[...], preferred_element_type=F32)
            + bm_ref[...], 0.0)

    return pl.pallas_call(
        body,
        grid=(m // bm,),
        in_specs=[
            pl.BlockSpec((bm, h), lambda i: (i, 0)),
            pl.BlockSpec((bm, de), lambda i: (i, 0)),
            pl.BlockSpec((de, h), lambda i: (0, 0)),
            pl.BlockSpec((1, h), lambda i: (0, 0)),
            pl.BlockSpec((h, h), lambda i: (0, 0)),
            pl.BlockSpec((1, h), lambda i: (0, 0)),
        ],
        out_specs=pl.BlockSpec((bm, h), lambda i: (i, 0)),
        out_shape=jax.ShapeDtypeStruct((m, h), F32),
    )(g, ea, w_e, b_e.reshape(1, h), w_m, b_m.reshape(1, h))


def _head(v, agg, w_u, b_u, w_o1, b_o1, w_o2p, b_o2p, bm):
    m, h = v.shape
    n1 = w_o1.shape[1]
    n2 = w_o2p.shape[1]

    def body(v_ref, a_ref, wu_ref, bu_ref, w1_ref, b1_ref, w2_ref, b2_ref,
             o_ref):
        x = jnp.maximum(
            jnp.dot(v_ref[...] + a_ref[...], wu_ref[...],
                    preferred_element_type=F32) + bu_ref[...], 0.0)
        x = jnp.maximum(
            jnp.dot(x, w1_ref[...], preferred_element_type=F32)
            + b1_ref[...], 0.0)
        o_ref[...] = (jnp.dot(x, w2_ref[...], preferred_element_type=F32)
                      + b2_ref[...])

    return pl.pallas_call(
        body,
        grid=(m // bm,),
        in_specs=[
            pl.BlockSpec((bm, h), lambda i: (i, 0)),
            pl.BlockSpec((bm, h), lambda i: (i, 0)),
            pl.BlockSpec((h, h), lambda i: (0, 0)),
            pl.BlockSpec((1, h), lambda i: (0, 0)),
            pl.BlockSpec((h, n1), lambda i: (0, 0)),
            pl.BlockSpec((1, n1), lambda i: (0, 0)),
            pl.BlockSpec((n1, n2), lambda i: (0, 0)),
            pl.BlockSpec((1, n2), lambda i: (0, 0)),
        ],
        out_specs=pl.BlockSpec((bm, n2), lambda i: (i, 0)),
        out_shape=jax.ShapeDtypeStruct((m, n2), F32),
    )(v, agg, w_u, b_u.reshape(1, h), w_o1, b_o1.reshape(1, n1),
      w_o2p, b_o2p.reshape(1, n2))


_NC = 2
_NS = 16


def _sc_gather(table, idx, chunk):
    n_rows, d = table.shape
    e = idx.shape[0]
    nw = _NC * _NS
    per_w = e // nw
    n_chunks = per_w // chunk
    mesh = plsc.VectorSubcoreMesh(core_axis_name="c", subcore_axis_name="s")

    @functools.partial(
        pl.kernel, mesh=mesh,
        out_type=jax.ShapeDtypeStruct((e, d), F32),
        scratch_types=[
            pltpu.VMEM((chunk,), jnp.int32),
            pltpu.VMEM((chunk, d), F32),
            pltpu.SemaphoreType.DMA,
        ],
    )
    def k(table_hbm, idx_hbm, out_hbm, idx_v, rows_v, sem):
        wid = lax.axis_index("s") * _NC + lax.axis_index("c")
        base0 = wid * per_w

        def body(i, carry):
            base = base0 + i * chunk
            pltpu.sync_copy(idx_hbm.at[pl.ds(base, chunk)], idx_v)
            pltpu.async_copy(table_hbm.at[idx_v], rows_v, sem).wait()
            pltpu.sync_copy(rows_v, out_hbm.at[pl.ds(base, chunk)])
            return carry

        lax.fori_loop(0, n_chunks, body, 0)

    return k(table, idx)


def _sc_scatter_add(rows, idx, zeros_half, chunk):
    e, d = rows.shape
    half = d // _NC
    r = zeros_half.shape[0]
    per_t = e // _NS
    n_chunks = per_t // chunk
    rows_per_t = r // _NS
    mesh = plsc.VectorSubcoreMesh(core_axis_name="c", subcore_axis_name="s")

    @functools.partial(
        pl.kernel, mesh=mesh,
        out_type=jax.ShapeDtypeStruct((r, d), F32),
        scratch_types=[
            pltpu.VMEM((chunk,), jnp.int32),
            pltpu.VMEM((chunk, half), F32),
            pltpu.VMEM_SHARED((r, half), F32),
            pltpu.SemaphoreType.DMA,
        ],
    )
    def k(rows_hbm, idx_hbm, zeros_hbm, out_hbm, idx_v, buf_v, acc_sh, sem):
        cid = lax.axis_index("c")
        sid = lax.axis_index("s")
        r0 = sid * rows_per_t
        pltpu.sync_copy(zeros_hbm.at[pl.ds(r0, rows_per_t)],
                        acc_sh.at[pl.ds(r0, rows_per_t)])
        plsc.subcore_barrier()

        def body(i, carry):
            base = sid * per_t + i * chunk
            pltpu.sync_copy(idx_hbm.at[pl.ds(base, chunk)], idx_v)
            pltpu.sync_copy(
                rows_hbm.at[pl.ds(base, chunk), pl.ds(cid * half, half)],
                buf_v)
            pltpu.sync_copy(buf_v, acc_sh.at[idx_v], add=True)
            return carry

        lax.fori_loop(0, n_chunks, body, 0)
        plsc.subcore_barrier()
        pltpu.sync_copy(
            acc_sh.at[pl.ds(r0, rows_per_t)],
            out_hbm.at[pl.ds(r0, rows_per_t), pl.ds(cid * half, half)])

    return k(rows, idx, zeros_half)


def kernel(constraint_features, variable_features, edge_attr,
           W_ce, b_ce, W_ve, b_ve, W_e, b_e,
           W_m1, b_m1, W_u1, b_u1, W_m2, b_m2, W_u2, b_u2,
           W_o1, b_o1, W_o2, b_o2,
           edge_index, graph_num):
    cons_idx = edge_index[0].astype(jnp.int32)
    var_idx = edge_index[1].astype(jnp.int32)
    n_cons = constraint_features.shape[0]
    h = W_ce.shape[1]

    c = _linrelu(constraint_features, W_ce, b_ce, bm=1000)
    v = _linrelu(variable_features, W_ve, b_ve, bm=1000)

    r_pad = ((n_cons + _NS * 8 - 1) // (_NS * 8)) * (_NS * 8)
    zeros_half = jnp.zeros((r_pad, h // _NC), F32)

    vg = _sc_gather(v, var_idx, chunk=200)
    m1 = _edge_stage(vg, edge_attr, W_e, b_e, W_m1, b_m1, bm=1000)
    agg_c = _sc_scatter_add(m1, cons_idx, zeros_half, chunk=200)[:n_cons]
    c = _addlinrelu(c, agg_c, W_u1, b_u1, bm=1000)

    cg = _sc_gather(c, cons_idx, chunk=200)
    m2 = _edge_stage(cg, edge_attr, W_e, b_e, W_m2, b_m2, bm=1000)
    agg_v = _sc_scatter_add(m2, var_idx, zeros_half, chunk=200)[:n_cons]

    w_o2p = jnp.pad(W_o2, ((0, 0), (0, 127)))
    b_o2p = jnp.pad(b_o2, (0, 127))
    out = _head(v, agg_v, W_u2, b_u2, W_o1, b_o1, w_o2p, b_o2p, bm=1000)
    return out[:, :1].reshape(-1, 1000, 1)


# speedup vs baseline: 2.6697x; 1.1843x over previous
import functools

import jax
import jax.numpy as jnp
from jax import lax
from jax.experimental import pallas as pl
from jax.experimental.pallas import tpu as pltpu
from jax.experimental.pallas import tpu_sc as plsc

F32 = jnp.float32


def _linrelu(x, w, b, bm):
    m, k = x.shape
    n = w.shape[1]

    def body(x_ref, w_ref, b_ref, o_ref):
        o_ref[...] = jnp.maximum(
            jnp.dot(x_ref[...], w_ref[...], preferred_element_type=F32)
            + b_ref[...], 0.0)

    return pl.pallas_call(
        body,
        grid=(m // bm,),
        in_specs=[
            pl.BlockSpec((bm, k), lambda i: (i, 0)),
            pl.BlockSpec((k, n), lambda i: (0, 0)),
            pl.BlockSpec((1, n), lambda i: (0, 0)),
        ],
        out_specs=pl.BlockSpec((bm, n), lambda i: (i, 0)),
        out_shape=jax.ShapeDtypeStruct((m, n), F32),
    )(x, w, b.reshape(1, n))


def _addlinrelu(x, y3, w, b, bm):
    m, k = x.shape
    n = w.shape[1]
    half = k // 2

    def body(x_ref, y_ref, w_ref, b_ref, o_ref):
        y = jnp.concatenate([y_ref[0, :, :], y_ref[1, :, :]], axis=-1)
        o_ref[...] = jnp.maximum(
            jnp.dot(x_ref[...] + y, w_ref[...],
                    preferred_element_type=F32) + b_ref[...], 0.0)

    return pl.pallas_call(
        body,
        grid=(m // bm,),
        in_specs=[
            pl.BlockSpec((bm, k), lambda i: (i, 0)),
            pl.BlockSpec((2, bm, k // 2), lambda i: (0, i, 0)),
            pl.BlockSpec((k, n), lambda i: (0, 0)),
            pl.BlockSpec((1, n), lambda i: (0, 0)),
        ],
        out_specs=pl.BlockSpec((bm, n), lambda i: (i, 0)),
        out_shape=jax.ShapeDtypeStruct((m, n), F32),
    )(x, y3, w, b.reshape(1, n))


def _edge_stage(g, ea, w_e, b_e, w_m, b_m, bm):
    m, h = g.shape
    de = ea.shape[1]
    half = h // 2

    def body(g_ref, ea_ref, we_ref, be_ref, wm_ref, bm_ref, o_ref):
        e = jnp.maximum(
            jnp.dot(ea_ref[...], we_ref[...], preferred_element_type=F32)
            + be_ref[...], 0.0)
        z = g_ref[...] + e
        o_ref[0, :, :] = jnp.maximum(
            jnp.dot(z, wm_ref[:, :half], preferred_element_type=F32)
            + bm_ref[:, :half], 0.0)
        o_ref[1, :, :] = jnp.maximum(
            jnp.dot(z, wm_ref[:, half:], preferred_element_type=F32)
            + bm_ref[:, half:], 0.0)

    return pl.pallas_call(
        body,
        grid=(m // bm,),
        in_specs=[
            pl.BlockSpec((bm, h), lambda i: (i, 0)),
            pl.BlockSpec((bm, de), lambda i: (i, 0)),
            pl.BlockSpec((de, h), lambda i: (0, 0)),
            pl.BlockSpec((1, h), lambda i: (0, 0)),
            pl.BlockSpec((h, h), lambda i: (0, 0)),
            pl.BlockSpec((1, h), lambda i: (0, 0)),
        ],
        out_specs=pl.BlockSpec((2, bm, half), lambda i: (0, i, 0)),
        out_shape=jax.ShapeDtypeStruct((2, m, half), F32),
    )(g, ea, w_e, b_e.reshape(1, h), w_m, b_m.reshape(1, h))


def _head(v, agg3, w_u, b_u, w_o1, b_o1, w_o2p, b_o2p, bm):
    m, h = v.shape
    half = h // 2
    n1 = w_o1.shape[1]
    n2 = w_o2p.shape[1]

    def body(v_ref, a_ref, wu_ref, bu_ref, w1_ref, b1_ref, w2_ref, b2_ref,
             o_ref):
        agg = jnp.concatenate([a_ref[0, :, :], a_ref[1, :, :]], axis=-1)
        x = jnp.maximum(
            jnp.dot(v_ref[...] + agg, wu_ref[...],
                    preferred_element_type=F32) + bu_ref[...], 0.0)
        x = jnp.maximum(
            jnp.dot(x, w1_ref[...], preferred_element_type=F32)
            + b1_ref[...], 0.0)
        o_ref[...] = (jnp.dot(x, w2_ref[...], preferred_element_type=F32)
                      + b2_ref[...])

    return pl.pallas_call(
        body,
        grid=(m // bm,),
        in_specs=[
            pl.BlockSpec((bm, h), lambda i: (i, 0)),
            pl.BlockSpec((2, bm, h // 2), lambda i: (0, i, 0)),
            pl.BlockSpec((h, h), lambda i: (0, 0)),
            pl.BlockSpec((1, h), lambda i: (0, 0)),
            pl.BlockSpec((h, n1), lambda i: (0, 0)),
            pl.BlockSpec((1, n1), lambda i: (0, 0)),
            pl.BlockSpec((n1, n2), lambda i: (0, 0)),
            pl.BlockSpec((1, n2), lambda i: (0, 0)),
        ],
        out_specs=pl.BlockSpec((bm, n2), lambda i: (i, 0)),
        out_shape=jax.ShapeDtypeStruct((m, n2), F32),
    )(v, agg3, w_u, b_u.reshape(1, h), w_o1, b_o1.reshape(1, n1),
      w_o2p, b_o2p.reshape(1, n2))


_NC = 2
_NS = 16


def _sc_gather(table, idx, chunk):
    n_rows, d = table.shape
    e = idx.shape[0]
    nw = _NC * _NS
    per_w = e // nw
    n_chunks = per_w // chunk
    mesh = plsc.VectorSubcoreMesh(core_axis_name="c", subcore_axis_name="s")

    @functools.partial(
        pl.kernel, mesh=mesh,
        out_type=jax.ShapeDtypeStruct((e, d), F32),
        scratch_types=[
            pltpu.VMEM((chunk,), jnp.int32),
            pltpu.VMEM((chunk,), jnp.int32),
            pltpu.VMEM((chunk, d), F32),
            pltpu.VMEM((chunk, d), F32),
            pltpu.SemaphoreType.DMA,
            pltpu.SemaphoreType.DMA,
        ],
    )
    def k(table_hbm, idx_hbm, out_hbm, idx0, idx1, rows0, rows1, sem0, sem1):
        wid = lax.axis_index("s") * _NC + lax.axis_index("c")
        base0 = wid * per_w
        pltpu.sync_copy(idx_hbm.at[pl.ds(base0, chunk)], idx0)
        pltpu.async_copy(table_hbm.at[idx0], rows0, sem0)

        def body(j, carry):
            c0 = 2 * j
            c1 = 2 * j + 1

            @pl.when(c1 < n_chunks)
            def _():
                b = base0 + c1 * chunk
                pltpu.sync_copy(idx_hbm.at[pl.ds(b, chunk)], idx1)
                pltpu.async_copy(table_hbm.at[idx1], rows1, sem1)

            pltpu.make_async_copy(table_hbm.at[idx0], rows0, sem0).wait()
            pltpu.sync_copy(rows0, out_hbm.at[pl.ds(base0 + c0 * chunk,
                                                    chunk)])

            @pl.when(c0 + 2 < n_chunks)
            def _():
                b = base0 + (c0 + 2) * chunk
                pltpu.sync_copy(idx_hbm.at[pl.ds(b, chunk)], idx0)
                pltpu.async_copy(table_hbm.at[idx0], rows0, sem0)

            @pl.when(c1 < n_chunks)
            def _():
                pltpu.make_async_copy(table_hbm.at[idx1], rows1, sem1).wait()
                pltpu.sync_copy(rows1, out_hbm.at[pl.ds(base0 + c1 * chunk,
                                                        chunk)])

            return carry

        lax.fori_loop(0, (n_chunks + 1) // 2, body, 0)

    return k(table, idx)


def _sc_scatter_add(rows3, idx, zeros_half, chunk):
    _, e, half = rows3.shape
    r = zeros_half.shape[0]
    n_chunks = e // chunk
    rows_per_t = r // _NS
    n_pairs = (n_chunks + 2 * _NS - 1) // (2 * _NS)
    mesh = plsc.VectorSubcoreMesh(core_axis_name="c", subcore_axis_name="s")

    @functools.partial(
        pl.kernel, mesh=mesh,
        out_type=jax.ShapeDtypeStruct((_NC, r, half), F32),
        scratch_types=[
            pltpu.VMEM((chunk,), jnp.int32),
            pltpu.VMEM((chunk,), jnp.int32),
            pltpu.VMEM((chunk, half), F32),
            pltpu.VMEM((chunk, half), F32),
            pltpu.VMEM_SHARED((r, half), F32),
            pltpu.SemaphoreType.DMA,
            pltpu.SemaphoreType.DMA,
        ],
    )
    def k(rows_hbm, idx_hbm, zeros_hbm, out_hbm,
          idx0, idx1, buf0, buf1, acc_sh, sem0, sem1):
        cid = lax.axis_index("c")
        sid = lax.axis_index("s")
        r0 = sid * rows_per_t

        def rows_at(c):
            return rows_hbm.at[cid, pl.ds(c * chunk, chunk)]

        pltpu.async_copy(rows_at(sid), buf0, sem0)
        pltpu.sync_copy(zeros_hbm.at[pl.ds(r0, rows_per_t)],
                        acc_sh.at[pl.ds(r0, rows_per_t)])
        plsc.subcore_barrier()

        def body(j, carry):
            c0 = sid + _NS * (2 * j)
            c1 = sid + _NS * (2 * j + 1)

            @pl.when(c1 < n_chunks)
            def _():
                pltpu.async_copy(rows_at(c1), buf1, sem1)

            @pl.when(c0 < n_chunks)
            def _():
                pltpu.sync_copy(idx_hbm.at[pl.ds(c0 * chunk, chunk)], idx0)
                pltpu.make_async_copy(rows_at(c0), buf0, sem0).wait()
                pltpu.sync_copy(buf0, acc_sh.at[idx0], add=True)

            @pl.when(c0 + 2 * _NS < n_chunks)
            def _():
                pltpu.async_copy(rows_at(c0 + 2 * _NS), buf0, sem0)

            @pl.when(c1 < n_chunks)
            def _():
                pltpu.sync_copy(idx_hbm.at[pl.ds(c1 * chunk, chunk)], idx1)
                pltpu.make_async_copy(rows_at(c1), buf1, sem1).wait()
                pltpu.sync_copy(buf1, acc_sh.at[idx1], add=True)

            return carry

        lax.fori_loop(0, n_pairs, body, 0)
        plsc.subcore_barrier()
        pltpu.sync_copy(
            acc_sh.at[pl.ds(r0, rows_per_t)],
            out_hbm.at[cid, pl.ds(r0, rows_per_t)])

    return k(rows3, idx, zeros_half)


def kernel(constraint_features, variable_features, edge_attr,
           W_ce, b_ce, W_ve, b_ve, W_e, b_e,
           W_m1, b_m1, W_u1, b_u1, W_m2, b_m2, W_u2, b_u2,
           W_o1, b_o1, W_o2, b_o2,
           edge_index, graph_num):
    cons_idx = edge_index[0].astype(jnp.int32)
    var_idx = edge_index[1].astype(jnp.int32)
    n_cons = constraint_features.shape[0]
    h = W_ce.shape[1]

    c = _linrelu(constraint_features, W_ce, b_ce, bm=1000)
    v = _linrelu(variable_features, W_ve, b_ve, bm=1000)

    r_pad = ((n_cons + _NS * 8 - 1) // (_NS * 8)) * (_NS * 8)
    zeros_half = jnp.zeros((r_pad, h // _NC), F32)

    vg = _sc_gather(v, var_idx, chunk=200)
    m1 = _edge_stage(vg, edge_attr, W_e, b_e, W_m1, b_m1, bm=1000)
    agg_c = _sc_scatter_add(m1, cons_idx, zeros_half, chunk=160)[:, :n_cons]
    c = _addlinrelu(c, agg_c, W_u1, b_u1, bm=1000)

    cg = _sc_gather(c, cons_idx, chunk=200)
    m2 = _edge_stage(cg, edge_attr, W_e, b_e, W_m2, b_m2, bm=1000)
    agg_v = _sc_scatter_add(m2, var_idx, zeros_half, chunk=160)[:, :n_cons]

    w_o2p = jnp.pad(W_o2, ((0, 0), (0, 127)))
    b_o2p = jnp.pad(b_o2, (0, 127))
    out = _head(v, agg_v, W_u2, b_u2, W_o1, b_o1, w_o2p, b_o2p, bm=1000)
    return out[:, :1].reshape(-1, 1000, 1)


# speedup vs baseline: 3.1150x; 1.1668x over previous
import functools

import jax
import jax.numpy as jnp
from jax import lax
from jax.experimental import pallas as pl
from jax.experimental.pallas import tpu as pltpu
from jax.experimental.pallas import tpu_sc as plsc

F32 = jnp.float32
U32 = jnp.int32


def _pack_f16(z):
    k = z.shape[-1] // 2
    i32 = jnp.int32

    def enc(x):
        u = jax.lax.bitcast_convert_type(x, i32)
        ur = u + i32(0xFFF) + ((u >> 13) & i32(1))
        h = (ur - i32(112 << 23)) >> 13
        h = jnp.where(ur < i32(113 << 23), i32(0), h)
        return jnp.minimum(h, i32(0x7BFF))

    return enc(z[:, :k]) | (enc(z[:, k:]) << 16)


def _unpack_f16(p):
    i32 = jnp.int32

    def dec(h):
        f = (h << 13) + i32(112 << 23)
        f = jnp.where(h == 0, i32(0), f)
        return jax.lax.bitcast_convert_type(f, F32)

    return jnp.concatenate(
        [dec(p & i32(0xFFFF)),
         dec(jax.lax.shift_right_logical(p, 16))], axis=-1)


def _linrelu(x, w, b, bm):
    m, k = x.shape
    n = w.shape[1]

    def body(x_ref, w_ref, b_ref, o_ref):
        o_ref[...] = jnp.maximum(
            jnp.dot(x_ref[...], w_ref[...], preferred_element_type=F32)
            + b_ref[...], 0.0)

    return pl.pallas_call(
        body,
        grid=(m // bm,),
        in_specs=[
            pl.BlockSpec((bm, k), lambda i: (i, 0)),
            pl.BlockSpec((k, n), lambda i: (0, 0)),
            pl.BlockSpec((1, n), lambda i: (0, 0)),
        ],
        out_specs=pl.BlockSpec((bm, n), lambda i: (i, 0)),
        out_shape=jax.ShapeDtypeStruct((m, n), F32),
    )(x, w, b.reshape(1, n))


def _linrelu_dual(x, w, b, bm):
    m, k = x.shape
    n = w.shape[1]

    def body(x_ref, w_ref, b_ref, o_ref, op_ref):
        z = jnp.maximum(
            jnp.dot(x_ref[...], w_ref[...], preferred_element_type=F32)
            + b_ref[...], 0.0)
        o_ref[...] = z
        op_ref[...] = _pack_f16(z)

    return pl.pallas_call(
        body,
        grid=(m // bm,),
        in_specs=[
            pl.BlockSpec((bm, k), lambda i: (i, 0)),
            pl.BlockSpec((k, n), lambda i: (0, 0)),
            pl.BlockSpec((1, n), lambda i: (0, 0)),
        ],
        out_specs=[
            pl.BlockSpec((bm, n), lambda i: (i, 0)),
            pl.BlockSpec((bm, n // 2), lambda i: (i, 0)),
        ],
        out_shape=[
            jax.ShapeDtypeStruct((m, n), F32),
            jax.ShapeDtypeStruct((m, n // 2), U32),
        ],
    )(x, w, b.reshape(1, n))


def _addlinrelu_pk(x, y3, w, b, bm):
    m, k = x.shape
    n = w.shape[1]

    def body(x_ref, y_ref, w_ref, b_ref, o_ref):
        y = jnp.concatenate([y_ref[0, :, :], y_ref[1, :, :]], axis=-1)
        z = jnp.maximum(
            jnp.dot(x_ref[...] + y, w_ref[...],
                    preferred_element_type=F32) + b_ref[...], 0.0)
        o_ref[...] = _pack_f16(z)

    return pl.pallas_call(
        body,
        grid=(m // bm,),
        in_specs=[
            pl.BlockSpec((bm, k), lambda i: (i, 0)),
            pl.BlockSpec((2, bm, k // 2), lambda i: (0, i, 0)),
            pl.BlockSpec((k, n), lambda i: (0, 0)),
            pl.BlockSpec((1, n), lambda i: (0, 0)),
        ],
        out_specs=pl.BlockSpec((bm, n // 2), lambda i: (i, 0)),
        out_shape=jax.ShapeDtypeStruct((m, n // 2), U32),
    )(x, y3, w, b.reshape(1, n))


def _edge_stage(g, ea, w_e, b_e, w_m, b_m, bm):
    m = g.shape[0]
    h = w_m.shape[0]
    de = ea.shape[1]
    half = h // 2

    def body(g_ref, ea_ref, we_ref, be_ref, wm_ref, bm_ref, o_ref):
        e = jnp.maximum(
            jnp.dot(ea_ref[...], we_ref[...], preferred_element_type=F32)
            + be_ref[...], 0.0)
        gf = _unpack_f16(g_ref[...])
        z = gf + e
        o_ref[0, :, :] = jnp.maximum(
            jnp.dot(z, wm_ref[:, :half], preferred_element_type=F32)
            + bm_ref[:, :half], 0.0)
        o_ref[1, :, :] = jnp.maximum(
            jnp.dot(z, wm_ref[:, half:], preferred_element_type=F32)
            + bm_ref[:, half:], 0.0)

    return pl.pallas_call(
        body,
        grid=(m // bm,),
        in_specs=[
            pl.BlockSpec((bm, half), lambda i: (i, 0)),
            pl.BlockSpec((bm, de), lambda i: (i, 0)),
            pl.BlockSpec((de, h), lambda i: (0, 0)),
            pl.BlockSpec((1, h), lambda i: (0, 0)),
            pl.BlockSpec((h, h), lambda i: (0, 0)),
            pl.BlockSpec((1, h), lambda i: (0, 0)),
        ],
        out_specs=pl.BlockSpec((2, bm, half), lambda i: (0, i, 0)),
        out_shape=jax.ShapeDtypeStruct((2, m, half), F32),
    )(g, ea, w_e, b_e.reshape(1, h), w_m, b_m.reshape(1, h))


def _head(v, agg3, w_u, b_u, w_o1, b_o1, w_o2p, b_o2p, bm):
    m, h = v.shape
    half = h // 2
    n1 = w_o1.shape[1]
    n2 = w_o2p.shape[1]

    def body(v_ref, a_ref, wu_ref, bu_ref, w1_ref, b1_ref, w2_ref, b2_ref,
             o_ref):
        agg = jnp.concatenate([a_ref[0, :, :], a_ref[1, :, :]], axis=-1)
        x = jnp.maximum(
            jnp.dot(v_ref[...] + agg, wu_ref[...],
                    preferred_element_type=F32) + bu_ref[...], 0.0)
        x = jnp.maximum(
            jnp.dot(x, w1_ref[...], preferred_element_type=F32)
            + b1_ref[...], 0.0)
        o_ref[...] = (jnp.dot(x, w2_ref[...], preferred_element_type=F32)
                      + b2_ref[...])

    return pl.pallas_call(
        body,
        grid=(m // bm,),
        in_specs=[
            pl.BlockSpec((bm, h), lambda i: (i, 0)),
            pl.BlockSpec((2, bm, h // 2), lambda i: (0, i, 0)),
            pl.BlockSpec((h, h), lambda i: (0, 0)),
            pl.BlockSpec((1, h), lambda i: (0, 0)),
            pl.BlockSpec((h, n1), lambda i: (0, 0)),
            pl.BlockSpec((1, n1), lambda i: (0, 0)),
            pl.BlockSpec((n1, n2), lambda i: (0, 0)),
            pl.BlockSpec((1, n2), lambda i: (0, 0)),
        ],
        out_specs=pl.BlockSpec((bm, n2), lambda i: (i, 0)),
        out_shape=jax.ShapeDtypeStruct((m, n2), F32),
    )(v, agg3, w_u, b_u.reshape(1, h), w_o1, b_o1.reshape(1, n1),
      w_o2p, b_o2p.reshape(1, n2))


_NC = 2
_NS = 16


def _sc_gather(table, idx, chunk):
    e = idx.shape[0]
    half = table.shape[1]
    nw = _NC * _NS
    n_chunks = e // chunk
    n_pairs = (n_chunks + 2 * nw - 1) // (2 * nw)
    mesh = plsc.VectorSubcoreMesh(core_axis_name="c", subcore_axis_name="s")

    @functools.partial(
        pl.kernel, mesh=mesh,
        out_type=jax.ShapeDtypeStruct((e, half), U32),
        scratch_types=[
            pltpu.VMEM((chunk,), jnp.int32),
            pltpu.VMEM((chunk,), jnp.int32),
            pltpu.VMEM((chunk, half), U32),
            pltpu.VMEM((chunk, half), U32),
            pltpu.SemaphoreType.DMA,
            pltpu.SemaphoreType.DMA,
        ],
    )
    def k(table_hbm, idx_hbm, out_hbm, idx0, idx1, rows0, rows1, sem0, sem1):
        wid = lax.axis_index("s") * _NC + lax.axis_index("c")

        def start(c, idx_v, rows_v, sem):
            pltpu.sync_copy(idx_hbm.at[pl.ds(c * chunk, chunk)], idx_v)
            pltpu.async_copy(table_hbm.at[idx_v], rows_v, sem)

        def drain(c, idx_v, rows_v, sem):
            pltpu.make_async_copy(table_hbm.at[idx_v], rows_v, sem).wait()
            pltpu.sync_copy(rows_v, out_hbm.at[pl.ds(c * chunk, chunk)])

        start(wid, idx0, rows0, sem0)

        def body(j, carry):
            c0 = wid + nw * (2 * j)
            c1 = wid + nw * (2 * j + 1)

            @pl.when(c1 < n_chunks)
            def _():
                start(c1, idx1, rows1, sem1)

            @pl.when(c0 < n_chunks)
            def _():
                drain(c0, idx0, rows0, sem0)

            @pl.when(c0 + 2 * nw < n_chunks)
            def _():
                start(c0 + 2 * nw, idx0, rows0, sem0)

            @pl.when(c1 < n_chunks)
            def _():
                drain(c1, idx1, rows1, sem1)

            return carry

        lax.fori_loop(0, n_pairs, body, 0)

    return k(table, idx)


def _sc_scatter_add(rows3, idx, zeros_half, chunk):
    _, e, half = rows3.shape
    r = zeros_half.shape[0]
    n_chunks = e // chunk
    rows_per_t = r // _NS
    n_pairs = (n_chunks + 2 * _NS - 1) // (2 * _NS)
    mesh = plsc.VectorSubcoreMesh(core_axis_name="c", subcore_axis_name="s")

    @functools.partial(
        pl.kernel, mesh=mesh,
        out_type=jax.ShapeDtypeStruct((_NC, r, half), F32),
        scratch_types=[
            pltpu.VMEM((chunk,), jnp.int32),
            pltpu.VMEM((chunk,), jnp.int32),
            pltpu.VMEM((chunk, half), F32),
            pltpu.VMEM((chunk, half), F32),
            pltpu.VMEM_SHARED((r, half), F32),
            pltpu.SemaphoreType.DMA,
            pltpu.SemaphoreType.DMA,
        ],
    )
    def k(rows_hbm, idx_hbm, zeros_hbm, out_hbm,
          idx0, idx1, buf0, buf1, acc_sh, sem0, sem1):
        cid = lax.axis_index("c")
        sid = lax.axis_index("s")
        r0 = sid * rows_per_t

        def rows_at(c):
            return rows_hbm.at[cid, pl.ds(c * chunk, chunk)]

        pltpu.async_copy(rows_at(sid), buf0, sem0)
        pltpu.sync_copy(zeros_hbm.at[pl.ds(r0, rows_per_t)],
                        acc_sh.at[pl.ds(r0, rows_per_t)])
        plsc.subcore_barrier()

        def body(j, carry):
            c0 = sid + _NS * (2 * j)
            c1 = sid + _NS * (2 * j + 1)

            @pl.when(c1 < n_chunks)
            def _():
                pltpu.async_copy(rows_at(c1), buf1, sem1)

            @pl.when(c0 < n_chunks)
            def _():
                pltpu.sync_copy(idx_hbm.at[pl.ds(c0 * chunk, chunk)], idx0)
                pltpu.make_async_copy(rows_at(c0), buf0, sem0).wait()
                pltpu.sync_copy(buf0, acc_sh.at[idx0], add=True)

            @pl.when(c0 + 2 * _NS < n_chunks)
            def _():
                pltpu.async_copy(rows_at(c0 + 2 * _NS), buf0, sem0)

            @pl.when(c1 < n_chunks)
            def _():
                pltpu.sync_copy(idx_hbm.at[pl.ds(c1 * chunk, chunk)], idx1)
                pltpu.make_async_copy(rows_at(c1), buf1, sem1).wait()
                pltpu.sync_copy(buf1, acc_sh.at[idx1], add=True)

            return carry

        lax.fori_loop(0, n_pairs, body, 0)
        plsc.subcore_barrier()
        pltpu.sync_copy(
            acc_sh.at[pl.ds(r0, rows_per_t)],
            out_hbm.at[cid, pl.ds(r0, rows_per_t)])

    return k(rows3, idx, zeros_half)


def kernel(constraint_features, variable_features, edge_attr,
           W_ce, b_ce, W_ve, b_ve, W_e, b_e,
           W_m1, b_m1, W_u1, b_u1, W_m2, b_m2, W_u2, b_u2,
           W_o1, b_o1, W_o2, b_o2,
           edge_index, graph_num):
    cons_idx = edge_index[0].astype(jnp.int32)
    var_idx = edge_index[1].astype(jnp.int32)
    n_cons = constraint_features.shape[0]
    h = W_ce.shape[1]

    c = _linrelu(constraint_features, W_ce, b_ce, bm=1000)
    v, v3 = _linrelu_dual(variable_features, W_ve, b_ve, bm=1000)

    r_pad = ((n_cons + _NS * 8 - 1) // (_NS * 8)) * (_NS * 8)
    zeros_half = jnp.zeros((r_pad, h // _NC), F32)

    vg = _sc_gather(v3, var_idx, chunk=320)
    m1 = _edge_stage(vg, edge_attr, W_e, b_e, W_m1, b_m1, bm=1000)
    agg_c = _sc_scatter_add(m1, cons_idx, zeros_half, chunk=160)[:, :n_cons]
    c2p = _addlinrelu_pk(c, agg_c, W_u1, b_u1, bm=1000)

    cg = _sc_gather(c2p, cons_idx, chunk=320)
    m2 = _edge_stage(cg, edge_attr, W_e, b_e, W_m2, b_m2, bm=1000)
    agg_v = _sc_scatter_add(m2, var_idx, zeros_half, chunk=160)[:, :n_cons]

    w_o2p = jnp.pad(W_o2, ((0, 0), (0, 127)))
    b_o2p = jnp.pad(b_o2, (0, 127))
    out = _head(v, agg_v, W_u2, b_u2, W_o1, b_o1, w_o2p, b_o2p, bm=1000)
    return out[:, :1].reshape(-1, 1000, 1)
